# baseline (speedup 1.0000x reference)
import functools

import jax
import jax.numpy as jnp
from jax import lax
from jax.experimental import pallas as pl
from jax.experimental.pallas import tpu as pltpu
from jax.experimental.pallas import tpu_sc as plsc

S, B, W = 50, 1024, 21
V, C, E = 96, 50, 128
K = 5
T = W - K + 1
N = S * B
L = 16
NCHUNK = E // L

NC, NS = 2, 16
NW = NC * NS
WPW = N // NW
CHUNK = 160
NLOOPS = WPW // CHUNK


def _fold_body(table_ref, cw_ref, cb_ref, out_ref):
    tab = table_ref[...]
    for k in range(K):
        wk = cw_ref[k]
        mk = jax.lax.dot_general(tab, wk, (((1,), (0,)), ((), ())),
                                 preferred_element_type=jnp.float32)
        if k == 0:
            mk = mk + cb_ref[...]
        out_ref[k * V:(k + 1) * V, :] = mk.astype(jnp.bfloat16)


def _fold_tables(table, conv_w, conv_b):
    cw = jnp.transpose(conv_w, (2, 1, 0))
    cb = conv_b.reshape(1, E)
    return pl.pallas_call(
        _fold_body,
        out_shape=jax.ShapeDtypeStruct((K * V, E), jnp.bfloat16),
    )(table, cw, cb)


EP = E // 2
NPCH = EP // L


def _sc_conv_body(m_hbm, idx_hbm, out_hbm, m_v, idx_v, out_v):
    wid = lax.axis_index("s") * NC + lax.axis_index("c")
    base = wid * WPW
    pltpu.sync_copy(m_hbm, m_v)
    iota = lax.iota(jnp.int32, L)

    def chunk_body(ci, _):
        w0 = base + ci * CHUNK
        pltpu.sync_copy(idx_hbm.at[pl.ds(w0 * W, CHUNK * W)], idx_v)

        cvec = [iota + k * V * EP for k in range(K)]

        def word_body(wl, _):
            woff = wl * W

            def cbc(j):
                return plsc.load_gather(
                    idx_v, [jnp.broadcast_to(woff + j, (L,))])

            cw_ = [cbc(j) for j in range(K)]
            run = [jnp.zeros((2 * L,), jnp.bfloat16) for _ in range(NPCH)]
            for t in range(T):
                idxk = [cw_[k] + cvec[k] for k in range(K)]
                for ch in range(NPCH):
                    g = [plsc.bitcast(plsc.load_gather(m_v, [idxk[k]]),
                                      jnp.bfloat16) for k in range(K)]
                    acc = ((g[0] + g[1]) + (g[2] + g[3])) + g[4]
                    run[ch] = jnp.maximum(run[ch], acc)
                    if ch + 1 < NPCH:
                        idxk = [x + L for x in idxk]
                if t + 1 < T:
                    cw_ = cw_[1:] + [cbc(t + K)]
            for ch in range(NPCH):
                out_v[wl, pl.ds(ch * 2 * L, 2 * L)] = run[ch]
            return ()

        lax.fori_loop(0, CHUNK, word_body, ())
        pltpu.sync_copy(out_v, out_hbm.at[pl.ds(w0, CHUNK)])
        return ()

    lax.fori_loop(0, NLOOPS, chunk_body, ())


def _sc_conv(mcat, idxw):
    m_i32 = jax.lax.bitcast_convert_type(
        mcat.reshape(K * V, EP, 2), jnp.int32).reshape(K * V * EP)
    idx64 = idxw.reshape(N * W) * EP
    mesh = plsc.VectorSubcoreMesh(core_axis_name="c", subcore_axis_name="s")
    f = functools.partial(
        pl.kernel, mesh=mesh,
        out_type=jax.ShapeDtypeStruct((N, E), jnp.bfloat16),
        scratch_types=[
            pltpu.VMEM((K * V * EP,), jnp.int32),
            pltpu.VMEM((CHUNK * W,), jnp.int32),
            pltpu.VMEM((CHUNK, E), jnp.bfloat16),
        ],
        compiler_params=pltpu.CompilerParams(needs_layout_passes=False),
    )(_sc_conv_body)
    return f(m_i32, idx64)


def _hw_body(x_ref, wp_ref, bp_ref, wg_ref, bg_ref, out_ref):
    x = x_ref[...].astype(jnp.float32)
    proj = jax.lax.dot_general(x, wp_ref[...], (((1,), (1,)), ((), ())),
                               preferred_element_type=jnp.float32)
    proj = jnp.maximum(proj + bp_ref[...], 0.0)
    gate = jax.lax.dot_general(x, wg_ref[...], (((1,), (1,)), ((), ())),
                               preferred_element_type=jnp.float32)
    gate = jax.nn.sigmoid(gate + bg_ref[...])
    out_ref[...] = gate * proj + (1.0 - gate) * x


def _highway(x, w_proj, b_proj, w_gate, b_gate, n=2048):
    return pl.pallas_call(
        _hw_body,
        grid=(N // n,),
        in_specs=[
            pl.BlockSpec((n, E), lambda i: (i, 0)),
            pl.BlockSpec((E, E), lambda i: (0, 0)),
            pl.BlockSpec((1, E), lambda i: (0, 0)),
            pl.BlockSpec((E, E), lambda i: (0, 0)),
            pl.BlockSpec((1, E), lambda i: (0, 0)),
        ],
        out_specs=pl.BlockSpec((n, E), lambda i: (i, 0)),
        out_shape=jax.ShapeDtypeStruct((N, E), jnp.float32),
        compiler_params=pltpu.CompilerParams(
            dimension_semantics=("arbitrary",),
        ),
    )(x, w_proj, b_proj.reshape(1, E), w_gate, b_gate.reshape(1, E))


def kernel(input, table, conv_w, conv_b, W_proj, b_proj, W_gate, b_gate):
    idxw = jnp.transpose(input, (1, 0, 2)).reshape(N, W).astype(jnp.int32)
    mcat = _fold_tables(table, conv_w, conv_b)
    conv = _sc_conv(mcat, idxw)
    out = _highway(conv, W_proj, b_proj, W_gate, b_gate)
    return out.reshape(S, B, E)


# speedup vs baseline: 1.6699x; 1.6699x over previous
import functools

import jax
import jax.numpy as jnp
from jax import lax
from jax.experimental import pallas as pl
from jax.experimental.pallas import tpu as pltpu
from jax.experimental.pallas import tpu_sc as plsc

S, B, W = 50, 1024, 21
V, C, E = 96, 50, 128
K = 5
T = W - K + 1
N = S * B
L = 16
NCHUNK = E // L

NC, NS = 2, 16
NW = NC * NS
WPW = N // NW
CHUNK = 160
NLOOPS = WPW // CHUNK


def _fold_body(table_ref, cw_ref, cb_ref, out_ref):
    tab = table_ref[...]
    for k in range(K):
        wk = cw_ref[k]
        mk = jax.lax.dot_general(tab, wk, (((1,), (0,)), ((), ())),
                                 preferred_element_type=jnp.float32)
        if k == 0:
            mk = mk + cb_ref[...]
        out_ref[k * V:(k + 1) * V, :] = mk


def _fold_tables(table, conv_w, conv_b):
    cw = jnp.transpose(conv_w, (2, 1, 0))
    cb = conv_b.reshape(1, E)
    return pl.pallas_call(
        _fold_body,
        out_shape=jax.ShapeDtypeStruct((K * V, E), jnp.float32),
    )(table, cw, cb)


def _sc_conv_body(m_hbm, idx_hbm, out_hbm, m_v, idx_v, out_v):
    wid = lax.axis_index("s") * NC + lax.axis_index("c")
    base = wid * WPW
    pltpu.sync_copy(m_hbm, m_v)
    iota = lax.iota(jnp.int32, L)

    def chunk_body(ci, _):
        w0 = base + ci * CHUNK
        pltpu.sync_copy(idx_hbm.at[pl.ds(w0 * W, CHUNK * W)], idx_v)

        def word_body(wl, _):
            woff = wl * W
            cj = [plsc.load_gather(
                      idx_v, [jnp.broadcast_to(woff + j, (L,))]) * E
                  for j in range(W)]
            run = [jnp.zeros((L,), jnp.float32) for _ in range(NCHUNK)]
            for t in range(T):
                idxk = [cj[t + k] + (k * V * E) + iota for k in range(K)]
                for ch in range(NCHUNK):
                    acc = plsc.load_gather(m_v, [idxk[0]])
                    for k in range(1, K):
                        acc = acc + plsc.load_gather(m_v, [idxk[k]])
                    run[ch] = jnp.maximum(run[ch], acc)
                    if ch + 1 < NCHUNK:
                        for k in range(K):
                            idxk[k] = idxk[k] + L
            for ch in range(NCHUNK):
                out_v[wl, pl.ds(ch * L, L)] = run[ch]
            return ()

        lax.fori_loop(0, CHUNK, word_body, ())
        pltpu.sync_copy(out_v, out_hbm.at[pl.ds(w0, CHUNK)])
        return ()

    lax.fori_loop(0, NLOOPS, chunk_body, ())


def _sc_conv(mcat, idxw):
    mesh = plsc.VectorSubcoreMesh(core_axis_name="c", subcore_axis_name="s")
    f = functools.partial(
        pl.kernel, mesh=mesh,
        out_type=jax.ShapeDtypeStruct((N, E), jnp.float32),
        scratch_types=[
            pltpu.VMEM((K * V * E,), jnp.float32),
            pltpu.VMEM((CHUNK * W,), jnp.int32),
            pltpu.VMEM((CHUNK, E), jnp.float32),
        ],
        compiler_params=pltpu.CompilerParams(needs_layout_passes=False),
    )(_sc_conv_body)
    return f(mcat.reshape(K * V * E), idxw.reshape(N * W))


def _hw_body(x_ref, wp_ref, bp_ref, wg_ref, bg_ref, out_ref):
    x = x_ref[...]
    proj = jax.lax.dot_general(x, wp_ref[...], (((1,), (1,)), ((), ())),
                               preferred_element_type=jnp.float32)
    proj = jnp.maximum(proj + bp_ref[...], 0.0)
    gate = jax.lax.dot_general(x, wg_ref[...], (((1,), (1,)), ((), ())),
                               preferred_element_type=jnp.float32)
    gate = jax.nn.sigmoid(gate + bg_ref[...])
    out_ref[...] = gate * proj + (1.0 - gate) * x


def _highway(x, w_proj, b_proj, w_gate, b_gate, n=2048):
    return pl.pallas_call(
        _hw_body,
        grid=(N // n,),
        in_specs=[
            pl.BlockSpec((n, E), lambda i: (i, 0)),
            pl.BlockSpec((E, E), lambda i: (0, 0)),
            pl.BlockSpec((1, E), lambda i: (0, 0)),
            pl.BlockSpec((E, E), lambda i: (0, 0)),
            pl.BlockSpec((1, E), lambda i: (0, 0)),
        ],
        out_specs=pl.BlockSpec((n, E), lambda i: (i, 0)),
        out_shape=jax.ShapeDtypeStruct((N, E), jnp.float32),
        compiler_params=pltpu.CompilerParams(
            dimension_semantics=("arbitrary",),
        ),
    )(x, w_proj, b_proj.reshape(1, E), w_gate, b_gate.reshape(1, E))


def kernel(input, table, conv_w, conv_b, W_proj, b_proj, W_gate, b_gate):
    idxw = jnp.transpose(input, (1, 0, 2)).reshape(N, W).astype(jnp.int32)
    mcat = _fold_tables(table, conv_w, conv_b)
    conv = _sc_conv(mcat, idxw)
    out = _highway(conv, W_proj, b_proj, W_gate, b_gate)
    return out.reshape(S, B, E)


# speedup vs baseline: 2.8527x; 1.7083x over previous
import functools

import jax
import jax.numpy as jnp
from jax import lax
from jax.experimental import pallas as pl
from jax.experimental.pallas import tpu as pltpu
from jax.experimental.pallas import tpu_sc as plsc

S, B, W = 50, 1024, 21
V, C, E = 96, 50, 128
K = 5
T = W - K + 1
N = S * B
L = 16
NCHUNK = E // L

NC, NS = 2, 16
NW = NC * NS
WPW = N // NW
CHUNK = 160
NLOOPS = WPW // CHUNK


def _fold_body(table_ref, cw_ref, cb_ref, out_ref):
    tab = table_ref[...]
    for k in range(K):
        wk = cw_ref[k]
        mk = jax.lax.dot_general(tab, wk, (((1,), (0,)), ((), ())),
                                 preferred_element_type=jnp.float32)
        if k == 0:
            mk = mk + cb_ref[...]
        out_ref[k * V:(k + 1) * V, :] = mk


def _fold_tables(table, conv_w, conv_b):
    cw = jnp.transpose(conv_w, (2, 1, 0))
    cb = conv_b.reshape(1, E)
    return pl.pallas_call(
        _fold_body,
        out_shape=jax.ShapeDtypeStruct((K * V, E), jnp.float32),
    )(table, cw, cb)


EP = E // 2
NPCH = EP // L


def _sc_conv_body(m_hbm, idx_hbm, out_hbm, m_v, idx_v, out_v):
    wid = lax.axis_index("s") * NC + lax.axis_index("c")
    base = wid * WPW
    pltpu.sync_copy(m_hbm, m_v)
    iota = lax.iota(jnp.int32, L)
    himask = jnp.full((L,), -65536, jnp.int32)

    def unpack_lo(g):
        return plsc.bitcast(lax.shift_left(g, 16), jnp.float32)

    def unpack_hi(g):
        return plsc.bitcast(lax.bitwise_and(g, himask), jnp.float32)

    def chunk_body(ci, _):
        w0 = base + ci * CHUNK
        pltpu.sync_copy(idx_hbm.at[pl.ds(w0 * W, CHUNK * W)], idx_v)

        def word_body(wl, _):
            woff = wl * W
            cj = [plsc.load_gather(
                      idx_v, [jnp.broadcast_to(woff + j, (L,))])
                  for j in range(W)]
            runl = [jnp.zeros((L,), jnp.float32) for _ in range(NPCH)]
            runh = [jnp.zeros((L,), jnp.float32) for _ in range(NPCH)]
            for t in range(T):
                idxk = [cj[t + k] + (k * V * EP) + iota for k in range(K)]
                for ch in range(NPCH):
                    g = [plsc.load_gather(m_v, [idxk[k]]) for k in range(K)]
                    gl = [unpack_lo(x) for x in g]
                    gh = [unpack_hi(x) for x in g]
                    accl = ((gl[0] + gl[1]) + (gl[2] + gl[3])) + gl[4]
                    acch = ((gh[0] + gh[1]) + (gh[2] + gh[3])) + gh[4]
                    runl[ch] = jnp.maximum(runl[ch], accl)
                    runh[ch] = jnp.maximum(runh[ch], acch)
                    if ch + 1 < NPCH:
                        idxk = [x + L for x in idxk]
            for ch in range(NPCH):
                out_v[wl, pl.ds(ch * L, L)] = runl[ch]
                out_v[wl, pl.ds(EP + ch * L, L)] = runh[ch]
            return ()

        lax.fori_loop(0, CHUNK, word_body, ())
        pltpu.sync_copy(out_v, out_hbm.at[pl.ds(w0, CHUNK)])
        return ()

    lax.fori_loop(0, NLOOPS, chunk_body, ())


def _pack_table(mcat):
    lo = jax.lax.bitcast_convert_type(
        mcat[:, :EP].astype(jnp.bfloat16), jnp.uint16).astype(jnp.uint32)
    hi = jax.lax.bitcast_convert_type(
        mcat[:, EP:].astype(jnp.bfloat16), jnp.uint16).astype(jnp.uint32)
    packed = jax.lax.bitwise_or(
        lo, jax.lax.shift_left(hi, jnp.uint32(16)))
    return jax.lax.bitcast_convert_type(packed, jnp.int32)


def _sc_conv(mcat, idxw):
    m_i32 = _pack_table(mcat).reshape(K * V * EP)
    idx_s = idxw.reshape(N * W) * EP
    mesh = plsc.VectorSubcoreMesh(core_axis_name="c", subcore_axis_name="s")
    f = functools.partial(
        pl.kernel, mesh=mesh,
        out_type=jax.ShapeDtypeStruct((N, E), jnp.float32),
        scratch_types=[
            pltpu.VMEM((K * V * EP,), jnp.int32),
            pltpu.VMEM((CHUNK * W,), jnp.int32),
            pltpu.VMEM((CHUNK, E), jnp.float32),
        ],
        compiler_params=pltpu.CompilerParams(needs_layout_passes=False),
    )(_sc_conv_body)
    return f(m_i32, idx_s)


def _hw_body(x_ref, wp_ref, bp_ref, wg_ref, bg_ref, out_ref):
    x = x_ref[...]
    proj = jax.lax.dot_general(x, wp_ref[...], (((1,), (1,)), ((), ())),
                               preferred_element_type=jnp.float32)
    proj = jnp.maximum(proj + bp_ref[...], 0.0)
    gate = jax.lax.dot_general(x, wg_ref[...], (((1,), (1,)), ((), ())),
                               preferred_element_type=jnp.float32)
    gate = jax.nn.sigmoid(gate + bg_ref[...])
    out_ref[...] = gate * proj + (1.0 - gate) * x


def _highway(x, w_proj, b_proj, w_gate, b_gate, n=2048):
    return pl.pallas_call(
        _hw_body,
        grid=(N // n,),
        in_specs=[
            pl.BlockSpec((n, E), lambda i: (i, 0)),
            pl.BlockSpec((E, E), lambda i: (0, 0)),
            pl.BlockSpec((1, E), lambda i: (0, 0)),
            pl.BlockSpec((E, E), lambda i: (0, 0)),
            pl.BlockSpec((1, E), lambda i: (0, 0)),
        ],
        out_specs=pl.BlockSpec((n, E), lambda i: (i, 0)),
        out_shape=jax.ShapeDtypeStruct((N, E), jnp.float32),
        compiler_params=pltpu.CompilerParams(
            dimension_semantics=("arbitrary",),
        ),
    )(x, w_proj, b_proj.reshape(1, E), w_gate, b_gate.reshape(1, E))


def kernel(input, table, conv_w, conv_b, W_proj, b_proj, W_gate, b_gate):
    idxw = jnp.transpose(input, (1, 0, 2)).reshape(N, W).astype(jnp.int32)
    mcat = _fold_tables(table, conv_w, conv_b)
    conv = _sc_conv(mcat, idxw)
    out = _highway(conv, W_proj, b_proj, W_gate, b_gate)
    return out.reshape(S, B, E)


# speedup vs baseline: 3.3908x; 1.1886x over previous
import functools

import jax
import jax.numpy as jnp
from jax import lax
from jax.experimental import pallas as pl
from jax.experimental.pallas import tpu as pltpu
from jax.experimental.pallas import tpu_sc as plsc

S, B, W = 50, 1024, 21
V, C, E = 96, 50, 128
K = 5
T = W - K + 1
N = S * B
L = 16
NCHUNK = E // L

NC, NS = 2, 16
NW = NC * NS
WPW = N // NW
CHUNK = 400
NLOOPS = WPW // CHUNK


def _fold_body(table_ref, cw_ref, cb_ref, out_ref):
    tab = table_ref[...]
    for k in range(K):
        wk = cw_ref[k]
        mk = jax.lax.dot_general(tab, wk, (((1,), (0,)), ((), ())),
                                 preferred_element_type=jnp.float32)
        if k == 0:
            mk = mk + cb_ref[...]
        out_ref[k * V:(k + 1) * V, :] = mk


def _fold_tables(table, conv_w, conv_b):
    cw = jnp.transpose(conv_w, (2, 1, 0))
    cb = conv_b.reshape(1, E)
    return pl.pallas_call(
        _fold_body,
        out_shape=jax.ShapeDtypeStruct((K * V, E), jnp.float32),
    )(table, cw, cb)


EP = E // 2
NPCH = EP // L


def _sc_conv_body(m_hbm, idx_hbm, out_hbm, m_v, idx_v, out_v):
    wid = lax.axis_index("s") * NC + lax.axis_index("c")
    base = wid * WPW
    pltpu.sync_copy(m_hbm, m_v)
    iota = lax.iota(jnp.int32, L)
    def unpack_lo(g):
        return plsc.bitcast(lax.shift_left(g, 16), jnp.float32)

    def unpack_hi(g):
        return plsc.bitcast(g, jnp.float32)

    def chunk_body(ci, _):
        w0 = base + ci * CHUNK
        pltpu.sync_copy(idx_hbm.at[pl.ds(w0 * W, CHUNK * W)], idx_v)

        def word_body(wl, _):
            woff = wl * W
            cj = [plsc.load_gather(
                      idx_v, [jnp.broadcast_to(woff + j, (L,))])
                  for j in range(W)]
            runl = [jnp.zeros((L,), jnp.float32) for _ in range(NPCH)]
            runh = [jnp.zeros((L,), jnp.float32) for _ in range(NPCH)]
            for t in range(T):
                idxk = [cj[t + k] + (k * V * EP) + iota for k in range(K)]
                for ch in range(NPCH):
                    g = [plsc.load_gather(m_v, [idxk[k]]) for k in range(K)]
                    gl = [unpack_lo(x) for x in g]
                    gh = [unpack_hi(x) for x in g]
                    accl = ((gl[0] + gl[1]) + (gl[2] + gl[3])) + gl[4]
                    acch = ((gh[0] + gh[1]) + (gh[2] + gh[3])) + gh[4]
                    runl[ch] = jnp.maximum(runl[ch], accl)
                    runh[ch] = jnp.maximum(runh[ch], acch)
                    if ch + 1 < NPCH:
                        idxk = [x + L for x in idxk]
            for ch in range(NPCH):
                out_v[wl, pl.ds(ch * L, L)] = runl[ch]
                out_v[wl, pl.ds(EP + ch * L, L)] = runh[ch]
            return ()

        lax.fori_loop(0, CHUNK, word_body, ())
        pltpu.sync_copy(out_v, out_hbm.at[pl.ds(w0, CHUNK)])
        return ()

    lax.fori_loop(0, NLOOPS, chunk_body, ())


def _pack_table(mcat):
    lo = jax.lax.bitcast_convert_type(
        mcat[:, :EP].astype(jnp.bfloat16), jnp.uint16).astype(jnp.uint32)
    hi = jax.lax.bitcast_convert_type(
        mcat[:, EP:].astype(jnp.bfloat16), jnp.uint16).astype(jnp.uint32)
    packed = jax.lax.bitwise_or(
        lo, jax.lax.shift_left(hi, jnp.uint32(16)))
    return jax.lax.bitcast_convert_type(packed, jnp.int32)


def _sc_conv(mcat, idxw):
    m_i32 = _pack_table(mcat).reshape(K * V * EP)
    idx_s = idxw.reshape(N * W) * EP
    mesh = plsc.VectorSubcoreMesh(core_axis_name="c", subcore_axis_name="s")
    f = functools.partial(
        pl.kernel, mesh=mesh,
        out_type=jax.ShapeDtypeStruct((N, E), jnp.float32),
        scratch_types=[
            pltpu.VMEM((K * V * EP,), jnp.int32),
            pltpu.VMEM((CHUNK * W,), jnp.int32),
            pltpu.VMEM((CHUNK, E), jnp.float32),
        ],
        compiler_params=pltpu.CompilerParams(needs_layout_passes=False),
    )(_sc_conv_body)
    return f(m_i32, idx_s)


def _hw_body(x_ref, wp_ref, bp_ref, wg_ref, bg_ref, out_ref):
    x = x_ref[...]
    proj = jax.lax.dot_general(x, wp_ref[...], (((1,), (1,)), ((), ())),
                               preferred_element_type=jnp.float32)
    proj = jnp.maximum(proj + bp_ref[...], 0.0)
    gate = jax.lax.dot_general(x, wg_ref[...], (((1,), (1,)), ((), ())),
                               preferred_element_type=jnp.float32)
    gate = jax.nn.sigmoid(gate + bg_ref[...])
    out_ref[...] = gate * proj + (1.0 - gate) * x


def _highway(x, w_proj, b_proj, w_gate, b_gate, n=2048):
    return pl.pallas_call(
        _hw_body,
        grid=(N // n,),
        in_specs=[
            pl.BlockSpec((n, E), lambda i: (i, 0)),
            pl.BlockSpec((E, E), lambda i: (0, 0)),
            pl.BlockSpec((1, E), lambda i: (0, 0)),
            pl.BlockSpec((E, E), lambda i: (0, 0)),
            pl.BlockSpec((1, E), lambda i: (0, 0)),
        ],
        out_specs=pl.BlockSpec((n, E), lambda i: (i, 0)),
        out_shape=jax.ShapeDtypeStruct((N, E), jnp.float32),
        compiler_params=pltpu.CompilerParams(
            dimension_semantics=("arbitrary",),
        ),
    )(x, w_proj, b_proj.reshape(1, E), w_gate, b_gate.reshape(1, E))


def kernel(input, table, conv_w, conv_b, W_proj, b_proj, W_gate, b_gate):
    idxw = jnp.transpose(input, (1, 0, 2)).reshape(N, W).astype(jnp.int32)
    mcat = _fold_tables(table, conv_w, conv_b)
    conv = _sc_conv(mcat, idxw)
    out = _highway(conv, W_proj, b_proj, W_gate, b_gate)
    return out.reshape(S, B, E)


# speedup vs baseline: 3.3954x; 1.0014x over previous
import functools

import jax
import jax.numpy as jnp
from jax import lax
from jax.experimental import pallas as pl
from jax.experimental.pallas import tpu as pltpu
from jax.experimental.pallas import tpu_sc as plsc

S, B, W = 50, 1024, 21
V, C, E = 96, 50, 128
K = 5
T = W - K + 1
N = S * B
L = 16

NC, NS = 2, 16
NW = NC * NS
WPW = N // NW
CHUNK = 400
NLOOPS = WPW // CHUNK


def _fold_body(table_ref, cw_ref, cb_ref, out_ref):
    tab = table_ref[...]
    for k in range(K):
        wk = cw_ref[k]
        mk = jax.lax.dot_general(tab, wk, (((1,), (0,)), ((), ())),
                                 preferred_element_type=jnp.float32)
        if k == 0:
            mk = mk + cb_ref[...]
        out_ref[k * V:(k + 1) * V, :] = mk


def _fold_tables(table, conv_w, conv_b):
    cw = jnp.transpose(conv_w, (2, 1, 0))
    cb = conv_b.reshape(1, E)
    return pl.pallas_call(
        _fold_body,
        out_shape=jax.ShapeDtypeStruct((K * V, E), jnp.float32),
    )(table, cw, cb)


EP = E // 2
NPCH = EP // L


def _sc_conv_body(m_hbm, idx_hbm, out_hbm, m_v, idx_v, out_v):
    wid = lax.axis_index("s") * NC + lax.axis_index("c")
    base = wid * WPW
    pltpu.sync_copy(m_hbm, m_v)
    iota = lax.iota(jnp.int32, L)
    def unpack_lo(g):
        return plsc.bitcast(lax.shift_left(g, 16), jnp.float32)

    def unpack_hi(g):
        return plsc.bitcast(g, jnp.float32)

    def chunk_body(ci, _):
        w0 = base + ci * CHUNK
        pltpu.sync_copy(idx_hbm.at[pl.ds(w0 * W, CHUNK * W)], idx_v)

        def word_body(wl, _):
            woff = wl * W
            cj = [plsc.load_gather(
                      idx_v, [jnp.broadcast_to(woff + j, (L,))])
                  for j in range(W)]
            runl = [jnp.zeros((L,), jnp.float32) for _ in range(NPCH)]
            runh = [jnp.zeros((L,), jnp.float32) for _ in range(NPCH)]
            for t in range(T):
                idxk = [cj[t + k] + (k * V * EP) + iota for k in range(K)]
                for ch in range(NPCH):
                    g = [plsc.load_gather(m_v, [idxk[k]]) for k in range(K)]
                    gl = [unpack_lo(x) for x in g]
                    gh = [unpack_hi(x) for x in g]
                    accl = ((gl[0] + gl[1]) + (gl[2] + gl[3])) + gl[4]
                    acch = ((gh[0] + gh[1]) + (gh[2] + gh[3])) + gh[4]
                    runl[ch] = jnp.maximum(runl[ch], accl)
                    runh[ch] = jnp.maximum(runh[ch], acch)
                    if ch + 1 < NPCH:
                        idxk = [x + L for x in idxk]
            for ch in range(NPCH):
                out_v[wl, pl.ds(ch * L, L)] = runl[ch]
                out_v[wl, pl.ds(EP + ch * L, L)] = runh[ch]
            return ()

        lax.fori_loop(0, CHUNK, word_body, ())
        pltpu.sync_copy(out_v, out_hbm.at[pl.ds(w0, CHUNK)])
        return ()

    lax.fori_loop(0, NLOOPS, chunk_body, ())


def _pack_table(mcat):
    lo = jax.lax.bitcast_convert_type(
        mcat[:, :EP].astype(jnp.bfloat16), jnp.uint16).astype(jnp.uint32)
    hi = jax.lax.bitcast_convert_type(
        mcat[:, EP:].astype(jnp.bfloat16), jnp.uint16).astype(jnp.uint32)
    packed = jax.lax.bitwise_or(
        lo, jax.lax.shift_left(hi, jnp.uint32(16)))
    return jax.lax.bitcast_convert_type(packed, jnp.int32)


def _sc_conv(mcat, idxw):
    m_i32 = _pack_table(mcat).reshape(K * V * EP)
    idx_s = idxw.reshape(N * W) * EP
    mesh = plsc.VectorSubcoreMesh(core_axis_name="c", subcore_axis_name="s")
    f = functools.partial(
        pl.kernel, mesh=mesh,
        out_type=jax.ShapeDtypeStruct((N, E), jnp.float32),
        scratch_types=[
            pltpu.VMEM((K * V * EP,), jnp.int32),
            pltpu.VMEM((CHUNK * W,), jnp.int32),
            pltpu.VMEM((CHUNK, E), jnp.float32),
        ],
        compiler_params=pltpu.CompilerParams(needs_layout_passes=False),
    )(_sc_conv_body)
    return f(m_i32, idx_s)


def _hw_body(x_ref, wp_ref, bp_ref, wg_ref, bg_ref, out_ref):
    x = x_ref[...]
    proj = jax.lax.dot_general(x, wp_ref[...], (((1,), (1,)), ((), ())),
                               preferred_element_type=jnp.float32)
    proj = jnp.maximum(proj + bp_ref[...], 0.0)
    gate = jax.lax.dot_general(x, wg_ref[...], (((1,), (1,)), ((), ())),
                               preferred_element_type=jnp.float32)
    gate = jax.nn.sigmoid(gate + bg_ref[...])
    out_ref[...] = gate * proj + (1.0 - gate) * x


def _highway(x, w_proj, b_proj, w_gate, b_gate, n=2048):
    return pl.pallas_call(
        _hw_body,
        grid=(N // n,),
        in_specs=[
            pl.BlockSpec((n, E), lambda i: (i, 0)),
            pl.BlockSpec((E, E), lambda i: (0, 0)),
            pl.BlockSpec((1, E), lambda i: (0, 0)),
            pl.BlockSpec((E, E), lambda i: (0, 0)),
            pl.BlockSpec((1, E), lambda i: (0, 0)),
        ],
        out_specs=pl.BlockSpec((n, E), lambda i: (i, 0)),
        out_shape=jax.ShapeDtypeStruct((N, E), jnp.float32),
        compiler_params=pltpu.CompilerParams(
            dimension_semantics=("arbitrary",),
        ),
    )(x, w_proj, b_proj.reshape(1, E), w_gate, b_gate.reshape(1, E))


def kernel(input, table, conv_w, conv_b, W_proj, b_proj, W_gate, b_gate):
    idxw = jnp.transpose(input, (1, 0, 2)).reshape(N, W).astype(jnp.int32)
    mcat = _fold_tables(table, conv_w, conv_b)
    conv = _sc_conv(mcat, idxw)
    out = _highway(conv, W_proj, b_proj, W_gate, b_gate)
    return out.reshape(S, B, E)
